# Initial kernel scaffold; baseline (speedup 1.0000x reference)
import jax
import jax.numpy as jnp
from jax.experimental import pallas as pl


def kernel(iword, owords, nwords, emb_i, emb_o):
    raise NotImplementedError("write your pallas kernel here")


# speedup vs baseline: 2.9777x; 2.9777x over previous
import functools

import jax
import jax.numpy as jnp
from jax import lax
from jax.experimental import pallas as pl
from jax.experimental.pallas import tpu as pltpu
from jax.experimental.pallas import tpu_sc as plsc

_NC = 2
_NS = 16
_NW = _NC * _NS


@functools.lru_cache(maxsize=None)
def _make_gather(V, D, NI, NO, NN):
    ni = NI // _NW
    no = NO // _NW
    nn = NN // _NW
    CH = 640
    assert no == CH and nn % CH == 0
    n_chunks = nn // CH
    mesh = plsc.VectorSubcoreMesh(core_axis_name="c", subcore_axis_name="s")

    @functools.partial(
        pl.kernel, mesh=mesh,
        out_type=[
            jax.ShapeDtypeStruct((NI, D), jnp.float32),
            jax.ShapeDtypeStruct((NO, D), jnp.float32),
            jax.ShapeDtypeStruct((NN, D), jnp.float32),
        ],
        scratch_types=[
            pltpu.VMEM((CH,), jnp.int32),
            pltpu.VMEM((CH, D), jnp.float32),
            pltpu.SemaphoreType.DMA,
        ],
        compiler_params=pltpu.CompilerParams(use_tc_tiling_on_sc=False),
    )
    def gather(emb_i, emb_o, iw, ow, nw, iv_out, ov_out, nv_out,
               idx_v, rows_v, sem):
        wid = lax.axis_index("s") * _NC + lax.axis_index("c")
        b0 = wid * ni
        pltpu.sync_copy(iw.at[pl.ds(b0, ni)], idx_v.at[pl.ds(0, ni)])
        pltpu.async_copy(emb_i.at[idx_v.at[pl.ds(0, ni)]],
                         rows_v.at[pl.ds(0, ni)], sem).wait()
        pltpu.sync_copy(rows_v.at[pl.ds(0, ni)], iv_out.at[pl.ds(b0, ni)])
        o0 = wid * no
        pltpu.sync_copy(ow.at[pl.ds(o0, no)], idx_v)
        pltpu.async_copy(emb_o.at[idx_v], rows_v, sem).wait()
        pltpu.sync_copy(rows_v, ov_out.at[pl.ds(o0, no)])
        def body(k, carry):
            s0 = pl.multiple_of(wid * nn + k * CH, 8)
            pltpu.sync_copy(nw.at[pl.ds(s0, CH)], idx_v)
            pltpu.async_copy(emb_o.at[idx_v], rows_v, sem).wait()
            pltpu.sync_copy(rows_v, nv_out.at[pl.ds(s0, CH)])
            return carry
        lax.fori_loop(0, n_chunks, body, 0)

    return gather


def _log_sigmoid(x):
    return jnp.minimum(x, 0.0) - jnp.log1p(jnp.exp(-jnp.abs(x)))


@functools.lru_cache(maxsize=None)
def _make_loss(B, C, NTOT, D, BB=64):
    nb = B // BB
    scale = -1.0 / (B * C)

    def body(iv_ref, ov_ref, nv_ref, out_ref):
        step = pl.program_id(0)
        iv = iv_ref[...]
        ov = ov_ref[...]
        nv = nv_ref[...]
        osc = jnp.sum(ov * iv[:, None, :], axis=-1)
        nsc = -jnp.sum(nv * iv[:, None, :], axis=-1)
        part = jnp.sum(_log_sigmoid(osc)) + jnp.sum(_log_sigmoid(nsc))

        @pl.when(step == 0)
        def _():
            out_ref[...] = jnp.zeros((1, 1), jnp.float32)

        out_ref[...] += scale * jnp.full((1, 1), part, jnp.float32)

    return pl.pallas_call(
        body,
        grid=(nb,),
        in_specs=[
            pl.BlockSpec((BB, D), lambda i: (i, 0)),
            pl.BlockSpec((BB, C, D), lambda i: (i, 0, 0)),
            pl.BlockSpec((BB, NTOT, D), lambda i: (i, 0, 0)),
        ],
        out_specs=pl.BlockSpec((1, 1), lambda i: (0, 0)),
        out_shape=jax.ShapeDtypeStruct((1, 1), jnp.float32),
    )


def kernel(iword, owords, nwords, emb_i, emb_o):
    V, D = emb_i.shape
    B, C = owords.shape
    NTOT = nwords.shape[1]
    iwf = iword.astype(jnp.int32)
    owf = owords.reshape(-1).astype(jnp.int32)
    nwf = nwords.reshape(-1).astype(jnp.int32)
    iv, ov, nv = _make_gather(V, D, B, B * C, B * NTOT)(
        emb_i, emb_o, iwf, owf, nwf)
    out = _make_loss(B, C, NTOT, D)(
        iv, ov.reshape(B, C, D), nv.reshape(B, NTOT, D))
    return out.reshape(())


# speedup vs baseline: 8.2683x; 2.7767x over previous
import functools

import jax
import jax.numpy as jnp
from jax import lax
from jax.experimental import pallas as pl
from jax.experimental.pallas import tpu as pltpu
from jax.experimental.pallas import tpu_sc as plsc

_NC = 2
_NS = 16
_NW = _NC * _NS
_L = 16


@functools.lru_cache(maxsize=None)
def _make_sc_scores(V, D, B, C, NTOT):
    ni = B // _NW
    no = (B * C) // _NW
    nn = (B * NTOT) // _NW
    NQ = D // _L
    assert C * NTOT == NTOT * C and nn == ni * NTOT and no == ni * C
    mesh = plsc.VectorSubcoreMesh(core_axis_name="c", subcore_axis_name="s")

    @functools.partial(
        pl.kernel, mesh=mesh,
        out_type=[
            jax.ShapeDtypeStruct((B * C,), jnp.float32),
            jax.ShapeDtypeStruct((B * NTOT,), jnp.float32),
        ],
        scratch_types=[
            pltpu.VMEM((ni,), jnp.int32),
            pltpu.VMEM((no,), jnp.int32),
            pltpu.VMEM((nn,), jnp.int32),
            pltpu.VMEM((ni, D), jnp.float32),
            pltpu.VMEM((no, D), jnp.float32),
            pltpu.VMEM((NTOT, D), jnp.float32),
            pltpu.VMEM((NTOT, D), jnp.float32),
            pltpu.VMEM((nn + no + _L,), jnp.float32),
            pltpu.SemaphoreType.DMA,
            pltpu.SemaphoreType.DMA,
            pltpu.SemaphoreType.DMA,
        ],
        compiler_params=pltpu.CompilerParams(
            use_tc_tiling_on_sc=False, needs_layout_passes=False),
    )
    def sgns_sc(emb_i, emb_o, iw, ow, nw, osc_out, nsc_out,
                iwi, owi, nwi, iv_v, ow_v, nv0, nv1, sc_v,
                semp, sem0, sem1):
        wid = lax.axis_index("s") * _NC + lax.axis_index("c")
        pltpu.sync_copy(iw.at[pl.ds(wid * ni, ni)], iwi)
        pltpu.sync_copy(ow.at[pl.ds(wid * no, no)], owi)
        pltpu.sync_copy(nw.at[pl.ds(wid * nn, nn)], nwi)
        pltpu.async_copy(emb_i.at[iwi], iv_v, semp).wait()
        pltpu.async_copy(emb_o.at[owi], ow_v, semp).wait()
        pltpu.async_copy(emb_o.at[nwi.at[pl.ds(0, NTOT)]], nv0, sem0)

        lane = lax.broadcasted_iota(jnp.int32, (_L,), 0)

        def dots_group(rows_ref, rbase, count, ivq, sbase):
            score = jnp.zeros((_L,), jnp.float32)
            for u in range(count):
                r = rbase + u
                p = rows_ref[r, pl.ds(0, _L)] * ivq[0]
                for q in range(1, NQ):
                    p += rows_ref[r, pl.ds(q * _L, _L)] * ivq[q]
                score = jnp.where(lane == u, jnp.sum(p), score)
            sc_v[pl.ds(sbase, _L)] = score

        def ivregs(b):
            return [iv_v[b, pl.ds(q * _L, _L)] for q in range(NQ)]

        def half(b, nv_cur, sem_cur, nv_nxt, sem_nxt, nxt_b, has_next):
            @pl.when(has_next)
            def _():
                pltpu.async_copy(
                    emb_o.at[nwi.at[pl.ds(nxt_b * NTOT, NTOT)]], nv_nxt,
                    sem_nxt)
            ivq = ivregs(b)
            ob = b * C
            for g0 in range(0, C, _L):
                dots_group(ow_v, ob + g0, min(_L, C - g0), ivq, nn + ob + g0)
            pltpu.make_async_copy(
                emb_o.at[nwi.at[pl.ds(0, NTOT)]], nv_cur, sem_cur).wait()

            def gbody(jj, c):
                dots_group(nv_cur, jj * _L, _L, ivq, b * NTOT + jj * _L)
                return c
            lax.fori_loop(0, NTOT // _L, gbody, 0)

        def pair(bb, c):
            b0 = 2 * bb
            half(b0, nv0, sem0, nv1, sem1, b0 + 1, True)
            half(b0 + 1, nv1, sem1, nv0, sem0, b0 + 2, bb < ni // 2 - 1)
            return c

        lax.fori_loop(0, ni // 2, pair, 0)
        pltpu.sync_copy(sc_v.at[pl.ds(nn, no)],
                        osc_out.at[pl.ds(wid * no, no)])
        pltpu.sync_copy(sc_v.at[pl.ds(0, nn)],
                        nsc_out.at[pl.ds(wid * nn, nn)])

    return sgns_sc


def _log_sigmoid(x):
    return jnp.minimum(x, 0.0) - jnp.log1p(jnp.exp(-jnp.abs(x)))


@functools.lru_cache(maxsize=None)
def _make_loss(B, C, NTOT):
    scale = -1.0 / (B * C)

    def body(osc_ref, nsc_ref, out_ref):
        part = (jnp.sum(_log_sigmoid(osc_ref[...]))
                + jnp.sum(_log_sigmoid(-nsc_ref[...])))
        out_ref[...] = scale * jnp.full((1, 1), part, jnp.float32)

    return pl.pallas_call(
        body,
        in_specs=[
            pl.BlockSpec((B, C), lambda: (0, 0)),
            pl.BlockSpec((B, NTOT), lambda: (0, 0)),
        ],
        out_specs=pl.BlockSpec((1, 1), lambda: (0, 0)),
        out_shape=jax.ShapeDtypeStruct((1, 1), jnp.float32),
    )


def kernel(iword, owords, nwords, emb_i, emb_o):
    V, D = emb_i.shape
    B, C = owords.shape
    NTOT = nwords.shape[1]
    iwf = iword.astype(jnp.int32)
    owf = owords.reshape(-1).astype(jnp.int32)
    nwf = nwords.reshape(-1).astype(jnp.int32)
    osc, nsc = _make_sc_scores(V, D, B, C, NTOT)(emb_i, emb_o, iwf, owf, nwf)
    out = _make_loss(B, C, NTOT)(osc.reshape(B, C), nsc.reshape(B, NTOT))
    return out.reshape(())
